# Initial kernel scaffold; baseline (speedup 1.0000x reference)
import jax
import jax.numpy as jnp
from jax.experimental import pallas as pl


def kernel(node_feature, edge_index, batch, W0, b0, Wc0, bc0, Wc1, bc1, Wp1, bp1, Wp2, bp2):
    raise NotImplementedError("write your pallas kernel here")


# speedup vs baseline: 15.7747x; 15.7747x over previous
import functools

import jax
import jax.numpy as jnp
from jax import lax
from jax.experimental import pallas as pl
from jax.experimental.pallas import tpu as pltpu
from jax.experimental.pallas import tpu_sc as plsc

_N = 10000
_E = 320000
_D = 128
_H = 128
_OUT = 32
_G = 64

_NCORE = 2
_NSUB = 16
_NW = _NCORE * _NSUB
_NPAD = 10240
_RPS = _NPAD // _NSUB
_CHUNK = 128
_NCHUNK = _E // _CHUNK
_CPW = (_NCHUNK + _NW - 1) // _NW

_mesh = plsc.VectorSubcoreMesh(core_axis_name="c", subcore_axis_name="s")


@functools.partial(
    pl.kernel,
    out_type=jax.ShapeDtypeStruct((_NCORE, _NPAD), jnp.float32),
    mesh=_mesh,
    scratch_types=[
        pltpu.VMEM((_CHUNK,), jnp.int32),
        pltpu.VMEM((_CHUNK,), jnp.float32),
        pltpu.VMEM_SHARED((_NPAD,), jnp.float32),
    ],
)
def _deg_pass(col_hbm, zero1_hbm, out_hbm, cidx, ones_v, acc):
    c = lax.axis_index("c")
    s = lax.axis_index("s")
    w = s * _NCORE + c
    for i in range(_CHUNK // 16):
        ones_v[pl.ds(i * 16, 16)] = jnp.ones((16,), jnp.float32)
    pltpu.sync_copy(zero1_hbm, acc.at[pl.ds(s * _RPS, _RPS)])
    plsc.subcore_barrier()

    def body(k, carry):
        j = w + k * _NW

        @pl.when(j < _NCHUNK)
        def _():
            pltpu.sync_copy(col_hbm.at[pl.ds(j * _CHUNK, _CHUNK)], cidx)
            pltpu.sync_copy(ones_v, acc.at[cidx], add=True)

        return carry

    lax.fori_loop(0, _CPW, body, 0)
    plsc.subcore_barrier()
    pltpu.sync_copy(acc.at[pl.ds(s * _RPS, _RPS)],
                    out_hbm.at[c, pl.ds(s * _RPS, _RPS)])


@functools.partial(
    pl.kernel,
    out_type=jax.ShapeDtypeStruct((_NCORE, _NPAD, _H), jnp.float32),
    mesh=_mesh,
    scratch_types=[
        pltpu.VMEM((_CHUNK,), jnp.int32),
        pltpu.VMEM((_CHUNK,), jnp.int32),
        pltpu.VMEM((_CHUNK, _H), jnp.float32),
        pltpu.VMEM_SHARED((_NPAD, _H), jnp.float32),
        pltpu.SemaphoreType.DMA,
    ],
)
def _edge_pass(row_hbm, col_hbm, y_hbm, zero2_hbm, out_hbm,
               ridx, cidx, rows, acc, sem):
    c = lax.axis_index("c")
    s = lax.axis_index("s")
    w = s * _NCORE + c
    pltpu.sync_copy(zero2_hbm, acc.at[pl.ds(s * _RPS, _RPS)])
    plsc.subcore_barrier()

    def body(k, carry):
        j = w + k * _NW

        @pl.when(j < _NCHUNK)
        def _():
            pltpu.sync_copy(row_hbm.at[pl.ds(j * _CHUNK, _CHUNK)], ridx)
            pltpu.sync_copy(col_hbm.at[pl.ds(j * _CHUNK, _CHUNK)], cidx)
            pltpu.async_copy(y_hbm.at[ridx], rows, sem).wait()
            pltpu.sync_copy(rows, acc.at[cidx], add=True)

        return carry

    lax.fori_loop(0, _CPW, body, 0)
    plsc.subcore_barrier()
    pltpu.sync_copy(acc.at[pl.ds(s * _RPS, _RPS)],
                    out_hbm.at[c, pl.ds(s * _RPS, _RPS)])


_R = 1000
_GRID = _N // _R


def _pre_body(degp, nf, w0, b0, wc0, xo, y0o):
    d = degp[...]
    dinv = lax.rsqrt(d[0] + d[1] + 1.0)
    x = lax.dot_general(nf[...], w0[...], (((1,), (1,)), ((), ())),
                        preferred_element_type=jnp.float32) + b0[...]
    xo[...] = x
    h0 = lax.dot_general(x, wc0[...], (((1,), (1,)), ((), ())),
                         preferred_element_type=jnp.float32)
    y0o[...] = dinv * h0


def _mid_body(degp, x, y0, s0p, bc0, wc1, h0ro, y1o):
    d = degp[...]
    dinv = lax.rsqrt(d[0] + d[1] + 1.0)
    sp = s0p[...]
    t = dinv * (sp[0] + sp[1] + y0[...]) + bc0[...]
    h0r = jnp.maximum(t, 0.0)
    h0ro[...] = h0r
    emb = jnp.concatenate([x[...], h0r], axis=1)
    h1 = lax.dot_general(emb, wc1[...], (((1,), (1,)), ((), ())),
                         preferred_element_type=jnp.float32)
    y1o[...] = dinv * h1


def _fin_body(degp, x, h0r, y1, s1p, bc1, bt, wp1, bp1, wp2, bp2,
              out, pooled):
    i = pl.program_id(0)
    d = degp[...]
    dinv = lax.rsqrt(d[0] + d[1] + 1.0)
    sp = s1p[...]
    t = dinv * (sp[0] + sp[1] + y1[...]) + bc1[...]
    h1r = jnp.maximum(t, 0.0)
    emb = jnp.concatenate([x[...], h0r[...], h1r], axis=1)
    seg = lax.broadcasted_iota(jnp.int32, (_R, _G), 1)
    onehot = jnp.where(bt[...] == seg, 1.0, 0.0).astype(jnp.float32)
    part = lax.dot_general(onehot, emb, (((0,), (0,)), ((), ())),
                           preferred_element_type=jnp.float32)

    @pl.when(i == 0)
    def _():
        pooled[...] = part

    @pl.when(i > 0)
    def _():
        pooled[...] = pooled[...] + part

    @pl.when(i == _GRID - 1)
    def _():
        p = pooled[...]
        h = lax.dot_general(p, wp1[...], (((1,), (1,)), ((), ())),
                            preferred_element_type=jnp.float32) + bp1[...]
        h = jnp.where(h > 0, h, 0.1 * h)
        o = lax.dot_general(h, wp2[...], (((1,), (1,)), ((), ())),
                            preferred_element_type=jnp.float32) + bp2[...]
        m = jnp.max(o, axis=1, keepdims=True)
        lse = jnp.log(jnp.sum(jnp.exp(o - m), axis=1, keepdims=True)) + m
        out[...] = o - lse


def kernel(node_feature, edge_index, batch, W0, b0, Wc0, bc0, Wc1, bc1,
           Wp1, bp1, Wp2, bp2):
    f32 = jnp.float32
    row = edge_index[0]
    col = edge_index[1]
    zero1 = jnp.zeros((_RPS,), f32)
    zero2 = jnp.zeros((_RPS, _H), f32)

    deg_p = _deg_pass(col, zero1)
    degp3 = deg_p.reshape(_NCORE, _NPAD, 1)

    dspec = pl.BlockSpec((_NCORE, _R, 1), lambda i: (0, i, 0))
    rspec = pl.BlockSpec((_R, _H), lambda i: (i, 0))
    sspec = pl.BlockSpec((_NCORE, _R, _H), lambda i: (0, i, 0))

    x, y0 = pl.pallas_call(
        _pre_body,
        grid=(_GRID,),
        in_specs=[
            dspec,
            pl.BlockSpec((_R, _D), lambda i: (i, 0)),
            pl.BlockSpec((_H, _D), lambda i: (0, 0)),
            pl.BlockSpec((1, _H), lambda i: (0, 0)),
            pl.BlockSpec((_H, _H), lambda i: (0, 0)),
        ],
        out_specs=[rspec, rspec],
        out_shape=[jax.ShapeDtypeStruct((_N, _H), f32)] * 2,
    )(degp3, node_feature, W0, b0.reshape(1, _H), Wc0)

    s0_p = _edge_pass(row, col, y0, zero2)

    h0r, y1 = pl.pallas_call(
        _mid_body,
        grid=(_GRID,),
        in_specs=[
            dspec, rspec, rspec, sspec,
            pl.BlockSpec((1, _H), lambda i: (0, 0)),
            pl.BlockSpec((_H, 2 * _H), lambda i: (0, 0)),
        ],
        out_specs=[rspec, rspec],
        out_shape=[jax.ShapeDtypeStruct((_N, _H), f32)] * 2,
    )(degp3, x, y0, s0_p, bc0.reshape(1, _H), Wc1)

    s1_p = _edge_pass(row, col, y1, zero2)

    out = pl.pallas_call(
        _fin_body,
        grid=(_GRID,),
        in_specs=[
            dspec, rspec, rspec, rspec, sspec,
            pl.BlockSpec((1, _H), lambda i: (0, 0)),
            pl.BlockSpec((_R, 1), lambda i: (i, 0)),
            pl.BlockSpec((_H, 3 * _H), lambda i: (0, 0)),
            pl.BlockSpec((1, _H), lambda i: (0, 0)),
            pl.BlockSpec((_OUT, _H), lambda i: (0, 0)),
            pl.BlockSpec((1, _OUT), lambda i: (0, 0)),
        ],
        out_specs=pl.BlockSpec((_G, _OUT), lambda i: (0, 0)),
        out_shape=jax.ShapeDtypeStruct((_G, _OUT), f32),
        scratch_shapes=[pltpu.VMEM((_G, 3 * _H), f32)],
    )(degp3, x, h0r, y1, s1_p, bc1.reshape(1, _H),
      batch.reshape(_N, 1), Wp1, bp1.reshape(1, _H), Wp2,
      bp2.reshape(1, _OUT))
    return out
